# Initial kernel scaffold; baseline (speedup 1.0000x reference)
import jax
import jax.numpy as jnp
from jax.experimental import pallas as pl


def kernel(x, edge_index):
    raise NotImplementedError("write your pallas kernel here")


# speedup vs baseline: 3.5740x; 3.5740x over previous
import dataclasses
import functools

import jax
import jax.numpy as jnp
from jax import lax
from jax.experimental import pallas as pl
from jax.experimental.pallas import tpu as pltpu
from jax.experimental.pallas import tpu_sc as plsc

N_NODES = 10000
N_EDGES = 320000
D_FEAT = 128

NC = 2
NS = 16
NW = NC * NS
CHUNK = 128
K = 80
E_PAD = NW * K * CHUNK
N_PAD = 10240
ROWS_PER_SUB = N_PAD // NS


def _sc_segment_sum(x, edges_r):
    mesh = plsc.VectorSubcoreMesh(core_axis_name="c", subcore_axis_name="s",
                                  num_cores=NC, num_subcores=NS)

    @functools.partial(
        pl.kernel,
        out_type=(
            jax.ShapeDtypeStruct((NC, N_PAD, D_FEAT), jnp.float32),
            jax.ShapeDtypeStruct((NW, N_PAD), jnp.float32),
        ),
        mesh=mesh,
        scratch_types=[
            pltpu.VMEM_SHARED((N_PAD, D_FEAT), jnp.float32),
            pltpu.VMEM((2, CHUNK), jnp.int32),
            pltpu.VMEM((CHUNK, D_FEAT), jnp.float32),
            pltpu.VMEM((N_PAD,), jnp.float32),
        ],
        compiler_params=dataclasses.replace(
            pltpu.CompilerParams(), needs_layout_passes=False),
    )
    def k(x_hbm, e_hbm, acc_hbm, cnt_hbm, acc_sh, idxb, rows, cnt_l):
        c = lax.axis_index("c")
        s = lax.axis_index("s")
        wid = c * NS + s

        @pl.loop(0, CHUNK)
        def _(r):
            @pl.loop(0, D_FEAT, step=16)
            def _(j):
                rows[r, pl.ds(j, 16)] = jnp.zeros((16,), jnp.float32)

        @pl.loop(0, N_PAD, step=16)
        def _(r):
            cnt_l[pl.ds(r, 16)] = jnp.zeros((16,), jnp.float32)

        base = s * ROWS_PER_SUB
        for b in range(ROWS_PER_SUB // CHUNK):
            pltpu.sync_copy(rows, acc_sh.at[pl.ds(base + b * CHUNK, CHUNK)])
        plsc.subcore_barrier()

        ones16 = jnp.ones((16,), jnp.float32)

        @pl.loop(0, K)
        def _(j):
            pltpu.sync_copy(e_hbm.at[wid, j], idxb)
            pltpu.sync_copy(x_hbm.at[idxb.at[0]], rows)
            pltpu.sync_copy(rows, acc_sh.at[idxb.at[1]], add=True)

            @pl.loop(0, CHUNK, step=16)
            def _(t):
                plsc.addupdate_scatter(cnt_l, [idxb[1, pl.ds(t, 16)]], ones16)

        plsc.subcore_barrier()

        pltpu.sync_copy(acc_sh.at[pl.ds(base, ROWS_PER_SUB)],
                        acc_hbm.at[c, pl.ds(base, ROWS_PER_SUB)])
        pltpu.sync_copy(cnt_l, cnt_hbm.at[wid])

    return k(x, edges_r)


def _combine_body(x_ref, a_ref, c_ref, o_ref):
    xb = x_ref[...]
    total = a_ref[0] + a_ref[1] + xb
    cnt = jnp.sum(c_ref[...], axis=1, keepdims=True) + 1.0
    o_ref[:, :D_FEAT] = xb
    o_ref[:, D_FEAT:] = total / cnt


def _tc_combine(x, acc, cnt):
    blk = 1000
    return pl.pallas_call(
        _combine_body,
        out_shape=jax.ShapeDtypeStruct((N_NODES, 2 * D_FEAT), jnp.float32),
        grid=(N_NODES // blk,),
        in_specs=[
            pl.BlockSpec((blk, D_FEAT), lambda i: (i, 0)),
            pl.BlockSpec((NC, blk, D_FEAT), lambda i: (0, i, 0)),
            pl.BlockSpec((blk, NW), lambda i: (i, 0)),
        ],
        out_specs=pl.BlockSpec((blk, 2 * D_FEAT), lambda i: (i, 0)),
    )(x, acc, cnt)


def kernel(x, edge_index):
    src = edge_index[0].astype(jnp.int32)
    dst = edge_index[1].astype(jnp.int32)
    pad = E_PAD - N_EDGES
    src_p = jnp.concatenate([src, jnp.zeros((pad,), jnp.int32)])
    dst_p = jnp.concatenate([dst, jnp.full((pad,), N_NODES, jnp.int32)])
    edges_r = jnp.stack(
        [src_p.reshape(NW, K, CHUNK), dst_p.reshape(NW, K, CHUNK)], axis=2)
    acc, cnt = _sc_segment_sum(x, edges_r)
    return _tc_combine(x, acc, cnt.T)


# speedup vs baseline: 3.9512x; 1.1055x over previous
import dataclasses
import functools

import jax
import jax.numpy as jnp
from jax import lax
from jax.experimental import pallas as pl
from jax.experimental.pallas import tpu as pltpu
from jax.experimental.pallas import tpu_sc as plsc

N_NODES = 10000
N_EDGES = 320000
D_FEAT = 128

NC = 2
NS = 16
NW = NC * NS
CHUNK = 80
K = 128
E_PAD = NW * K * CHUNK
N_PAD = 10240
ROWS_PER_SUB = N_PAD // NS


def _sc_segment_sum(x, edges_r):
    mesh = plsc.VectorSubcoreMesh(core_axis_name="c", subcore_axis_name="s",
                                  num_cores=NC, num_subcores=NS)

    @functools.partial(
        pl.kernel,
        out_type=(
            jax.ShapeDtypeStruct((NC, N_PAD, D_FEAT), jnp.float32),
            jax.ShapeDtypeStruct((NW, N_PAD), jnp.float32),
        ),
        mesh=mesh,
        scratch_types=[
            pltpu.VMEM_SHARED((N_PAD, D_FEAT), jnp.float32),
            pltpu.VMEM((2, CHUNK), jnp.int32),
            pltpu.VMEM((2, CHUNK), jnp.int32),
            pltpu.VMEM((2, CHUNK), jnp.int32),
            pltpu.VMEM((2, CHUNK), jnp.int32),
            pltpu.VMEM((CHUNK, D_FEAT), jnp.float32),
            pltpu.VMEM((CHUNK, D_FEAT), jnp.float32),
            pltpu.VMEM((N_PAD,), jnp.float32),
            pltpu.SemaphoreType.DMA,
            pltpu.SemaphoreType.DMA,
            pltpu.SemaphoreType.DMA,
            pltpu.SemaphoreType.DMA,
            pltpu.SemaphoreType.DMA,
            pltpu.SemaphoreType.DMA,
        ],
        compiler_params=dataclasses.replace(
            pltpu.CompilerParams(), needs_layout_passes=False),
    )
    def k(x_hbm, e_hbm, acc_hbm, cnt_hbm, acc_sh,
          i0, i1, i2, i3, rows0, rows1, cnt_l,
          si0, si1, si2, si3, sg0, sg1):
        c = lax.axis_index("c")
        s = lax.axis_index("s")
        wid = c * NS + s
        idxs = [i0, i1, i2, i3]
        isems = [si0, si1, si2, si3]
        rows = [rows0, rows1]
        gsems = [sg0, sg1]

        @pl.loop(0, CHUNK)
        def _(r):
            @pl.loop(0, D_FEAT, step=16)
            def _(j):
                rows0[r, pl.ds(j, 16)] = jnp.zeros((16,), jnp.float32)

        @pl.loop(0, N_PAD, step=16)
        def _(r):
            cnt_l[pl.ds(r, 16)] = jnp.zeros((16,), jnp.float32)

        base = s * ROWS_PER_SUB
        for b in range(ROWS_PER_SUB // CHUNK):
            pltpu.sync_copy(rows0, acc_sh.at[pl.ds(base + b * CHUNK, CHUNK)])
        plsc.subcore_barrier()

        ones16 = jnp.ones((16,), jnp.float32)

        pltpu.async_copy(e_hbm.at[wid, 0], i0, si0)
        pltpu.async_copy(e_hbm.at[wid, 1], i1, si1)
        pltpu.make_async_copy(e_hbm.at[wid, 0], i0, si0).wait()
        pltpu.async_copy(x_hbm.at[i0.at[0]], rows0, sg0)

        @pl.loop(0, K, step=4)
        def _(c0):
            for i in range(4):
                cc = c0 + i
                ib = idxs[i]
                rb = rows[i % 2]

                @pl.when(cc + 2 < K)
                def _():
                    pltpu.async_copy(e_hbm.at[wid, cc + 2],
                                     idxs[(i + 2) % 4], isems[(i + 2) % 4])

                pltpu.make_async_copy(x_hbm.at[ib.at[0]], rb,
                                      gsems[i % 2]).wait()

                @pl.when(cc + 1 < K)
                def _():
                    pltpu.make_async_copy(e_hbm.at[wid, cc + 1],
                                          idxs[(i + 1) % 4],
                                          isems[(i + 1) % 4]).wait()
                    pltpu.async_copy(x_hbm.at[idxs[(i + 1) % 4].at[0]],
                                     rows[(i + 1) % 2], gsems[(i + 1) % 2])

                pltpu.sync_copy(rb, acc_sh.at[ib.at[1]], add=True)

                @pl.loop(0, CHUNK, step=16)
                def _(t):
                    plsc.addupdate_scatter(cnt_l, [ib[1, pl.ds(t, 16)]],
                                           ones16)

        plsc.subcore_barrier()

        pltpu.sync_copy(acc_sh.at[pl.ds(base, ROWS_PER_SUB)],
                        acc_hbm.at[c, pl.ds(base, ROWS_PER_SUB)])
        pltpu.sync_copy(cnt_l, cnt_hbm.at[wid])

    return k(x, edges_r)


def _combine_body(x_ref, a_ref, c_ref, o_ref):
    xb = x_ref[...]
    total = a_ref[0] + a_ref[1] + xb
    cnt = jnp.sum(c_ref[...], axis=1, keepdims=True) + 1.0
    o_ref[:, :D_FEAT] = xb
    o_ref[:, D_FEAT:] = total / cnt


def _tc_combine(x, acc, cnt):
    blk = 1000
    return pl.pallas_call(
        _combine_body,
        out_shape=jax.ShapeDtypeStruct((N_NODES, 2 * D_FEAT), jnp.float32),
        grid=(N_NODES // blk,),
        in_specs=[
            pl.BlockSpec((blk, D_FEAT), lambda i: (i, 0)),
            pl.BlockSpec((NC, blk, D_FEAT), lambda i: (0, i, 0)),
            pl.BlockSpec((blk, NW), lambda i: (i, 0)),
        ],
        out_specs=pl.BlockSpec((blk, 2 * D_FEAT), lambda i: (i, 0)),
    )(x, acc, cnt)


def kernel(x, edge_index):
    src = edge_index[0].astype(jnp.int32)
    dst = edge_index[1].astype(jnp.int32)
    pad = E_PAD - N_EDGES
    src_p = jnp.concatenate([src, jnp.zeros((pad,), jnp.int32)])
    dst_p = jnp.concatenate([dst, jnp.full((pad,), N_NODES, jnp.int32)])
    edges_r = jnp.stack(
        [src_p.reshape(NW, K, CHUNK), dst_p.reshape(NW, K, CHUNK)], axis=2)
    acc, cnt = _sc_segment_sum(x, edges_r)
    return _tc_combine(x, acc, cnt.T)


# speedup vs baseline: 3.9589x; 1.0020x over previous
import dataclasses
import functools

import jax
import jax.numpy as jnp
from jax import lax
from jax.experimental import pallas as pl
from jax.experimental.pallas import tpu as pltpu
from jax.experimental.pallas import tpu_sc as plsc

N_NODES = 10000
N_EDGES = 320000
D_FEAT = 128

NC = 2
NS = 16
NW = NC * NS
CHUNK = 80
K = 128
E_PAD = NW * K * CHUNK
N_PAD = 10240
ROWS_PER_SUB = N_PAD // NS


def _sc_segment_sum(x, edges_r):
    mesh = plsc.VectorSubcoreMesh(core_axis_name="c", subcore_axis_name="s",
                                  num_cores=NC, num_subcores=NS)

    @functools.partial(
        pl.kernel,
        out_type=(
            jax.ShapeDtypeStruct((NC, N_PAD, D_FEAT), jnp.float32),
            jax.ShapeDtypeStruct((NW, N_PAD), jnp.float32),
        ),
        mesh=mesh,
        scratch_types=[
            pltpu.VMEM_SHARED((N_PAD, D_FEAT), jnp.float32),
            pltpu.VMEM((2, CHUNK), jnp.int32),
            pltpu.VMEM((2, CHUNK), jnp.int32),
            pltpu.VMEM((2, CHUNK), jnp.int32),
            pltpu.VMEM((2, CHUNK), jnp.int32),
            pltpu.VMEM((CHUNK, D_FEAT), jnp.float32),
            pltpu.VMEM((CHUNK, D_FEAT), jnp.float32),
            pltpu.VMEM((N_PAD,), jnp.float32),
            pltpu.SemaphoreType.DMA,
            pltpu.SemaphoreType.DMA,
            pltpu.SemaphoreType.DMA,
            pltpu.SemaphoreType.DMA,
            pltpu.SemaphoreType.DMA,
            pltpu.SemaphoreType.DMA,
        ],
        compiler_params=dataclasses.replace(
            pltpu.CompilerParams(), needs_layout_passes=False),
    )
    def k(x_hbm, e_hbm, acc_hbm, cnt_hbm, acc_sh,
          i0, i1, i2, i3, rows0, rows1, cnt_l,
          si0, si1, si2, si3, sg0, sg1):
        c = lax.axis_index("c")
        s = lax.axis_index("s")
        wid = c * NS + s
        idxs = [i0, i1, i2, i3]
        isems = [si0, si1, si2, si3]
        rows = [rows0, rows1]
        gsems = [sg0, sg1]

        @pl.loop(0, CHUNK)
        def _(r):
            @pl.loop(0, D_FEAT, step=16)
            def _(j):
                rows0[r, pl.ds(j, 16)] = jnp.zeros((16,), jnp.float32)

        @pl.loop(0, N_PAD, step=16)
        def _(r):
            cnt_l[pl.ds(r, 16)] = jnp.zeros((16,), jnp.float32)

        base = s * ROWS_PER_SUB
        for b in range(ROWS_PER_SUB // CHUNK):
            pltpu.sync_copy(rows0, acc_sh.at[pl.ds(base + b * CHUNK, CHUNK)])
        plsc.subcore_barrier()

        ones16 = jnp.ones((16,), jnp.float32)

        pltpu.async_copy(e_hbm.at[wid, 0], i0, si0)
        pltpu.async_copy(e_hbm.at[wid, 1], i1, si1)
        pltpu.make_async_copy(e_hbm.at[wid, 0], i0, si0).wait()
        pltpu.async_copy(x_hbm.at[i0.at[0]], rows0, sg0)

        @pl.loop(0, K, step=4)
        def _(c0):
            for i in range(4):
                cc = c0 + i
                ib = idxs[i]
                rb = rows[i % 2]

                @pl.when(cc + 2 < K)
                def _():
                    pltpu.async_copy(e_hbm.at[wid, cc + 2],
                                     idxs[(i + 2) % 4], isems[(i + 2) % 4])

                pltpu.make_async_copy(x_hbm.at[ib.at[0]], rb,
                                      gsems[i % 2]).wait()

                @pl.when(cc + 1 < K)
                def _():
                    pltpu.make_async_copy(e_hbm.at[wid, cc + 1],
                                          idxs[(i + 1) % 4],
                                          isems[(i + 1) % 4]).wait()
                    pltpu.async_copy(x_hbm.at[idxs[(i + 1) % 4].at[0]],
                                     rows[(i + 1) % 2], gsems[(i + 1) % 2])

                del rb

        plsc.subcore_barrier()

        pltpu.sync_copy(acc_sh.at[pl.ds(base, ROWS_PER_SUB)],
                        acc_hbm.at[c, pl.ds(base, ROWS_PER_SUB)])
        pltpu.sync_copy(cnt_l, cnt_hbm.at[wid])

    return k(x, edges_r)


def _combine_body(x_ref, a_ref, c_ref, o_ref):
    xb = x_ref[...]
    total = a_ref[0] + a_ref[1] + xb
    cnt = jnp.sum(c_ref[...], axis=1, keepdims=True) + 1.0
    o_ref[:, :D_FEAT] = xb
    o_ref[:, D_FEAT:] = total / cnt


def _tc_combine(x, acc, cnt):
    blk = 1000
    return pl.pallas_call(
        _combine_body,
        out_shape=jax.ShapeDtypeStruct((N_NODES, 2 * D_FEAT), jnp.float32),
        grid=(N_NODES // blk,),
        in_specs=[
            pl.BlockSpec((blk, D_FEAT), lambda i: (i, 0)),
            pl.BlockSpec((NC, blk, D_FEAT), lambda i: (0, i, 0)),
            pl.BlockSpec((blk, NW), lambda i: (i, 0)),
        ],
        out_specs=pl.BlockSpec((blk, 2 * D_FEAT), lambda i: (i, 0)),
    )(x, acc, cnt)


def kernel(x, edge_index):
    src = edge_index[0].astype(jnp.int32)
    dst = edge_index[1].astype(jnp.int32)
    pad = E_PAD - N_EDGES
    src_p = jnp.concatenate([src, jnp.zeros((pad,), jnp.int32)])
    dst_p = jnp.concatenate([dst, jnp.full((pad,), N_NODES, jnp.int32)])
    edges_r = jnp.stack(
        [src_p.reshape(NW, K, CHUNK), dst_p.reshape(NW, K, CHUNK)], axis=2)
    acc, cnt = _sc_segment_sum(x, edges_r)
    return _tc_combine(x, acc, cnt.T)


# speedup vs baseline: 8.8461x; 2.2345x over previous
import dataclasses
import functools

import jax
import jax.numpy as jnp
from jax import lax
from jax.experimental import pallas as pl
from jax.experimental.pallas import tpu as pltpu
from jax.experimental.pallas import tpu_sc as plsc

N_NODES = 10000
N_EDGES = 320000
D_FEAT = 128
DH = D_FEAT // 2

NC = 2
NS = 16
CHUNK = 80
K = 256
E_PAD = NS * K * CHUNK
N_PAD = 10240
ROWS_PER_SUB = N_PAD // NS
XROWS_PER_SUB = N_PAD // NS


def _sc_segment_sum(xs, edges_r):
    mesh = plsc.VectorSubcoreMesh(core_axis_name="c", subcore_axis_name="s",
                                  num_cores=NC, num_subcores=NS)

    @functools.partial(
        pl.kernel,
        out_type=(
            jax.ShapeDtypeStruct((NC, N_PAD, DH), jnp.float32),
            jax.ShapeDtypeStruct((NS, N_PAD), jnp.float32),
        ),
        mesh=mesh,
        scratch_types=[
            pltpu.VMEM_SHARED((N_PAD, DH), jnp.float32),
            pltpu.VMEM_SHARED((N_PAD, DH), jnp.float32),
            pltpu.VMEM((2, CHUNK), jnp.int32),
            pltpu.VMEM((2, CHUNK), jnp.int32),
            pltpu.VMEM((2, CHUNK), jnp.int32),
            pltpu.VMEM((2, CHUNK), jnp.int32),
            pltpu.VMEM((CHUNK, DH), jnp.float32),
            pltpu.VMEM((CHUNK, DH), jnp.float32),
            pltpu.VMEM((N_PAD,), jnp.float32),
            pltpu.SemaphoreType.DMA,
            pltpu.SemaphoreType.DMA,
            pltpu.SemaphoreType.DMA,
            pltpu.SemaphoreType.DMA,
            pltpu.SemaphoreType.DMA,
            pltpu.SemaphoreType.DMA,
        ],
        compiler_params=dataclasses.replace(
            pltpu.CompilerParams(), needs_layout_passes=False,
            use_tc_tiling_on_sc=False),
    )
    def k(xs_hbm, e_hbm, acc_hbm, cnt_hbm, xc_sh, acc_sh,
          i0, i1, i2, i3, rows0, rows1, cnt_l,
          si0, si1, si2, si3, sg0, sg1):
        c = lax.axis_index("c")
        s = lax.axis_index("s")
        idxs = [i0, i1, i2, i3]
        isems = [si0, si1, si2, si3]
        rows = [rows0, rows1]
        gsems = [sg0, sg1]

        xbase = s * XROWS_PER_SUB
        pltpu.sync_copy(xs_hbm.at[c, pl.ds(xbase, XROWS_PER_SUB)],
                        xc_sh.at[pl.ds(xbase, XROWS_PER_SUB)])

        @pl.loop(0, CHUNK)
        def _(r):
            @pl.loop(0, DH, step=16)
            def _(j):
                rows0[r, pl.ds(j, 16)] = jnp.zeros((16,), jnp.float32)

        @pl.loop(0, N_PAD, step=16)
        def _(r):
            cnt_l[pl.ds(r, 16)] = jnp.zeros((16,), jnp.float32)

        base = s * ROWS_PER_SUB
        for b in range(ROWS_PER_SUB // CHUNK):
            pltpu.sync_copy(rows0, acc_sh.at[pl.ds(base + b * CHUNK, CHUNK)])
        plsc.subcore_barrier()

        ones16 = jnp.ones((16,), jnp.float32)

        pltpu.async_copy(e_hbm.at[s, 0], i0, si0)
        pltpu.async_copy(e_hbm.at[s, 1], i1, si1)
        pltpu.make_async_copy(e_hbm.at[s, 0], i0, si0).wait()
        pltpu.async_copy(xc_sh.at[i0.at[0]], rows0, sg0)

        @pl.loop(0, K, step=4)
        def _(c0):
            for i in range(4):
                cc = c0 + i
                ib = idxs[i]
                rb = rows[i % 2]

                @pl.when(cc + 2 < K)
                def _():
                    pltpu.async_copy(e_hbm.at[s, cc + 2],
                                     idxs[(i + 2) % 4], isems[(i + 2) % 4])

                pltpu.make_async_copy(xc_sh.at[ib.at[0]], rb,
                                      gsems[i % 2]).wait()

                @pl.when(cc + 1 < K)
                def _():
                    pltpu.make_async_copy(e_hbm.at[s, cc + 1],
                                          idxs[(i + 1) % 4],
                                          isems[(i + 1) % 4]).wait()
                    pltpu.async_copy(xc_sh.at[idxs[(i + 1) % 4].at[0]],
                                     rows[(i + 1) % 2], gsems[(i + 1) % 2])

                pltpu.sync_copy(rb, acc_sh.at[ib.at[1]], add=True)

                @pl.when(c == 0)
                def _():
                    @pl.loop(0, CHUNK, step=16)
                    def _(t):
                        plsc.addupdate_scatter(cnt_l, [ib[1, pl.ds(t, 16)]],
                                               ones16)

        plsc.subcore_barrier()

        pltpu.sync_copy(acc_sh.at[pl.ds(base, ROWS_PER_SUB)],
                        acc_hbm.at[c, pl.ds(base, ROWS_PER_SUB)])

        @pl.when(c == 0)
        def _():
            pltpu.sync_copy(cnt_l, cnt_hbm.at[s])

    return k(xs, edges_r)


def _combine_body(x_ref, a_ref, c_ref, o_ref):
    xb = x_ref[...]
    inv = 1.0 / (jnp.sum(c_ref[...], axis=1, keepdims=True) + 1.0)
    o_ref[:, :D_FEAT] = xb
    o_ref[:, D_FEAT:D_FEAT + DH] = (a_ref[0] + xb[:, :DH]) * inv
    o_ref[:, D_FEAT + DH:] = (a_ref[1] + xb[:, DH:]) * inv


def _tc_combine(x, acc, cnt):
    blk = 1000
    return pl.pallas_call(
        _combine_body,
        out_shape=jax.ShapeDtypeStruct((N_NODES, 2 * D_FEAT), jnp.float32),
        grid=(N_NODES // blk,),
        in_specs=[
            pl.BlockSpec((blk, D_FEAT), lambda i: (i, 0)),
            pl.BlockSpec((NC, blk, DH), lambda i: (0, i, 0)),
            pl.BlockSpec((blk, NS), lambda i: (i, 0)),
        ],
        out_specs=pl.BlockSpec((blk, 2 * D_FEAT), lambda i: (i, 0)),
    )(x, acc, cnt)


def kernel(x, edge_index):
    src = edge_index[0].astype(jnp.int32)
    dst = edge_index[1].astype(jnp.int32)
    pad = E_PAD - N_EDGES
    src_p = jnp.concatenate([src, jnp.zeros((pad,), jnp.int32)])
    dst_p = jnp.concatenate([dst, jnp.full((pad,), N_NODES, jnp.int32)])
    edges_r = jnp.stack(
        [src_p.reshape(NS, K, CHUNK), dst_p.reshape(NS, K, CHUNK)], axis=2)
    xp = jnp.pad(x, ((0, N_PAD - N_NODES), (0, 0)))
    xs = jnp.stack([xp[:, :DH], xp[:, DH:]])
    acc, cnt = _sc_segment_sum(xs, edges_r)
    return _tc_combine(x, acc, cnt.T)


# speedup vs baseline: 9.6996x; 1.0965x over previous
import dataclasses
import functools

import jax
import jax.numpy as jnp
from jax import lax
from jax.experimental import pallas as pl
from jax.experimental.pallas import tpu as pltpu
from jax.experimental.pallas import tpu_sc as plsc

N_NODES = 10000
N_EDGES = 320000
D_FEAT = 128
DH = D_FEAT // 2

NC = 2
NS = 16
CHUNK = 128
K = 160
E_PAD = NS * K * CHUNK
N_PAD = 10240
ROWS_PER_SUB = N_PAD // NS
XROWS_PER_SUB = N_PAD // NS


def _sc_segment_sum(xs, edges_r):
    mesh = plsc.VectorSubcoreMesh(core_axis_name="c", subcore_axis_name="s",
                                  num_cores=NC, num_subcores=NS)

    @functools.partial(
        pl.kernel,
        out_type=(
            jax.ShapeDtypeStruct((NC, N_PAD, DH), jnp.float32),
            jax.ShapeDtypeStruct((NS, N_PAD), jnp.float32),
        ),
        mesh=mesh,
        scratch_types=[
            pltpu.VMEM_SHARED((N_PAD, DH), jnp.float32),
            pltpu.VMEM_SHARED((N_PAD, DH), jnp.float32),
            pltpu.VMEM((2, CHUNK), jnp.int32),
            pltpu.VMEM((2, CHUNK), jnp.int32),
            pltpu.VMEM((2, CHUNK), jnp.int32),
            pltpu.VMEM((2, CHUNK), jnp.int32),
            pltpu.VMEM((CHUNK, DH), jnp.float32),
            pltpu.VMEM((CHUNK, DH), jnp.float32),
            pltpu.VMEM((N_PAD,), jnp.float32),
            pltpu.SemaphoreType.DMA,
            pltpu.SemaphoreType.DMA,
            pltpu.SemaphoreType.DMA,
            pltpu.SemaphoreType.DMA,
            pltpu.SemaphoreType.DMA,
            pltpu.SemaphoreType.DMA,
        ],
        compiler_params=dataclasses.replace(
            pltpu.CompilerParams(), needs_layout_passes=False,
            use_tc_tiling_on_sc=False),
    )
    def k(xs_hbm, e_hbm, acc_hbm, cnt_hbm, xc_sh, acc_sh,
          i0, i1, i2, i3, rows0, rows1, cnt_l,
          si0, si1, si2, si3, sg0, sg1):
        c = lax.axis_index("c")
        s = lax.axis_index("s")
        idxs = [i0, i1, i2, i3]
        isems = [si0, si1, si2, si3]
        rows = [rows0, rows1]
        gsems = [sg0, sg1]

        xbase = s * XROWS_PER_SUB
        pltpu.sync_copy(xs_hbm.at[c, pl.ds(xbase, XROWS_PER_SUB)],
                        xc_sh.at[pl.ds(xbase, XROWS_PER_SUB)])

        @pl.loop(0, CHUNK)
        def _(r):
            @pl.loop(0, DH, step=16)
            def _(j):
                rows0[r, pl.ds(j, 16)] = jnp.zeros((16,), jnp.float32)

        @pl.loop(0, N_PAD, step=16)
        def _(r):
            cnt_l[pl.ds(r, 16)] = jnp.zeros((16,), jnp.float32)

        base = s * ROWS_PER_SUB
        for b in range(ROWS_PER_SUB // CHUNK):
            pltpu.sync_copy(rows0, acc_sh.at[pl.ds(base + b * CHUNK, CHUNK)])
        plsc.subcore_barrier()

        ones16 = jnp.ones((16,), jnp.float32)

        pltpu.async_copy(e_hbm.at[s, 0], i0, si0)
        pltpu.async_copy(e_hbm.at[s, 1], i1, si1)
        pltpu.make_async_copy(e_hbm.at[s, 0], i0, si0).wait()
        pltpu.async_copy(xc_sh.at[i0.at[0]], rows0, sg0)

        @pl.loop(0, K, step=4)
        def _(c0):
            for i in range(4):
                cc = c0 + i
                ib = idxs[i]
                rb = rows[i % 2]

                @pl.when(cc + 2 < K)
                def _():
                    pltpu.async_copy(e_hbm.at[s, cc + 2],
                                     idxs[(i + 2) % 4], isems[(i + 2) % 4])

                pltpu.make_async_copy(xc_sh.at[ib.at[0]], rb,
                                      gsems[i % 2]).wait()

                @pl.when(cc + 1 < K)
                def _():
                    pltpu.make_async_copy(e_hbm.at[s, cc + 1],
                                          idxs[(i + 1) % 4],
                                          isems[(i + 1) % 4]).wait()
                    pltpu.async_copy(xc_sh.at[idxs[(i + 1) % 4].at[0]],
                                     rows[(i + 1) % 2], gsems[(i + 1) % 2])

                pltpu.sync_copy(rb, acc_sh.at[ib.at[1]], add=True)

                @pl.when(c == 0)
                def _():
                    @pl.loop(0, CHUNK, step=16)
                    def _(t):
                        plsc.addupdate_scatter(cnt_l, [ib[1, pl.ds(t, 16)]],
                                               ones16)

        plsc.subcore_barrier()

        pltpu.sync_copy(acc_sh.at[pl.ds(base, ROWS_PER_SUB)],
                        acc_hbm.at[c, pl.ds(base, ROWS_PER_SUB)])

        @pl.when(c == 0)
        def _():
            pltpu.sync_copy(cnt_l, cnt_hbm.at[s])

    return k(xs, edges_r)


def _combine_body(x_ref, a_ref, c_ref, o_ref):
    xb = x_ref[...]
    inv = 1.0 / (jnp.sum(c_ref[...], axis=1, keepdims=True) + 1.0)
    o_ref[:, :D_FEAT] = xb
    o_ref[:, D_FEAT:D_FEAT + DH] = (a_ref[0] + xb[:, :DH]) * inv
    o_ref[:, D_FEAT + DH:] = (a_ref[1] + xb[:, DH:]) * inv


def _tc_combine(x, acc, cnt):
    blk = 1000
    return pl.pallas_call(
        _combine_body,
        out_shape=jax.ShapeDtypeStruct((N_NODES, 2 * D_FEAT), jnp.float32),
        grid=(N_NODES // blk,),
        in_specs=[
            pl.BlockSpec((blk, D_FEAT), lambda i: (i, 0)),
            pl.BlockSpec((NC, blk, DH), lambda i: (0, i, 0)),
            pl.BlockSpec((blk, NS), lambda i: (i, 0)),
        ],
        out_specs=pl.BlockSpec((blk, 2 * D_FEAT), lambda i: (i, 0)),
    )(x, acc, cnt)


def kernel(x, edge_index):
    src = edge_index[0].astype(jnp.int32)
    dst = edge_index[1].astype(jnp.int32)
    pad = E_PAD - N_EDGES
    src_p = jnp.concatenate([src, jnp.zeros((pad,), jnp.int32)])
    dst_p = jnp.concatenate([dst, jnp.full((pad,), N_NODES, jnp.int32)])
    edges_r = jnp.stack(
        [src_p.reshape(NS, K, CHUNK), dst_p.reshape(NS, K, CHUNK)], axis=2)
    xp = jnp.pad(x, ((0, N_PAD - N_NODES), (0, 0)))
    xs = jnp.stack([xp[:, :DH], xp[:, DH:]])
    acc, cnt = _sc_segment_sum(xs, edges_r)
    return _tc_combine(x, acc, cnt.T)


# speedup vs baseline: 11.4309x; 1.1785x over previous
import dataclasses
import functools

import jax
import jax.numpy as jnp
from jax import lax
from jax.experimental import pallas as pl
from jax.experimental.pallas import tpu as pltpu
from jax.experimental.pallas import tpu_sc as plsc

N_NODES = 10000
N_EDGES = 320000
D_FEAT = 128
DH = D_FEAT // 2

NC = 2
NS = 16
CHUNK = 128
E_TILE = N_EDGES // NS
K = E_TILE // CHUNK
TAIL = E_TILE - K * CHUNK
N_PAD = 10240
ROWS_PER_SUB = N_PAD // NS


def _sc_segment_sum(x, e2):
    mesh = plsc.VectorSubcoreMesh(core_axis_name="c", subcore_axis_name="s",
                                  num_cores=NC, num_subcores=NS)

    @functools.partial(
        pl.kernel,
        out_type=(
            jax.ShapeDtypeStruct((NC, N_PAD, DH), jnp.float32),
            jax.ShapeDtypeStruct((NS, N_PAD), jnp.float32),
        ),
        mesh=mesh,
        scratch_types=[
            pltpu.VMEM_SHARED((N_PAD, DH), jnp.float32),
            pltpu.VMEM_SHARED((N_PAD, DH), jnp.float32),
            pltpu.VMEM((2, CHUNK), jnp.int32),
            pltpu.VMEM((2, CHUNK), jnp.int32),
            pltpu.VMEM((2, CHUNK), jnp.int32),
            pltpu.VMEM((2, CHUNK), jnp.int32),
            pltpu.VMEM((CHUNK, DH), jnp.float32),
            pltpu.VMEM((CHUNK, DH), jnp.float32),
            pltpu.VMEM((N_PAD,), jnp.float32),
            pltpu.SemaphoreType.DMA,
            pltpu.SemaphoreType.DMA,
            pltpu.SemaphoreType.DMA,
            pltpu.SemaphoreType.DMA,
            pltpu.SemaphoreType.DMA,
            pltpu.SemaphoreType.DMA,
        ],
        compiler_params=dataclasses.replace(
            pltpu.CompilerParams(), needs_layout_passes=False,
            use_tc_tiling_on_sc=False),
    )
    def k(x_hbm, e_hbm, acc_hbm, cnt_hbm, xc_sh, acc_sh,
          i0, i1, i2, i3, rows0, rows1, cnt_l,
          si0, si1, si2, si3, sg0, sg1):
        c = lax.axis_index("c")
        s = lax.axis_index("s")
        idxs = [i0, i1, i2, i3]
        isems = [si0, si1, si2, si3]
        rows = [rows0, rows1]
        gsems = [sg0, sg1]
        ebase = s * E_TILE
        col = c * DH

        @pl.when(s < NS - 1)
        def _():
            pltpu.sync_copy(
                x_hbm.at[pl.ds(s * 640, 640), pl.ds(col, DH)],
                xc_sh.at[pl.ds(s * 640, 640)])

        @pl.when(s == NS - 1)
        def _():
            pltpu.sync_copy(
                x_hbm.at[pl.ds(9600, 400), pl.ds(col, DH)],
                xc_sh.at[pl.ds(9600, 400)])

        @pl.loop(0, CHUNK)
        def _(r):
            @pl.loop(0, DH, step=16)
            def _(j):
                rows0[r, pl.ds(j, 16)] = jnp.zeros((16,), jnp.float32)

        @pl.loop(0, N_PAD, step=16)
        def _(r):
            cnt_l[pl.ds(r, 16)] = jnp.zeros((16,), jnp.float32)

        base = s * ROWS_PER_SUB
        for b in range(ROWS_PER_SUB // CHUNK):
            pltpu.sync_copy(rows0, acc_sh.at[pl.ds(base + b * CHUNK, CHUNK)])
        plsc.subcore_barrier()

        ones16 = jnp.ones((16,), jnp.float32)

        def load_idx(j, ib, sem):
            off = ebase + j * CHUNK
            pltpu.async_copy(e_hbm.at[0, pl.ds(off, CHUNK)], ib.at[0], sem)
            pltpu.async_copy(e_hbm.at[1, pl.ds(off, CHUNK)], ib.at[1], sem)

        def wait_idx(ib, sem):
            pltpu.make_async_copy(e_hbm.at[0, pl.ds(0, CHUNK)], ib.at[0],
                                  sem).wait()
            pltpu.make_async_copy(e_hbm.at[1, pl.ds(0, CHUNK)], ib.at[1],
                                  sem).wait()

        load_idx(0, i0, si0)
        load_idx(1, i1, si1)
        wait_idx(i0, si0)
        pltpu.async_copy(xc_sh.at[i0.at[0]], rows0, sg0)

        @pl.loop(0, K, step=4)
        def _(c0):
            for i in range(4):
                cc = c0 + i
                ib = idxs[i]
                rb = rows[i % 2]

                @pl.when(cc + 2 < K)
                def _():
                    load_idx(cc + 2, idxs[(i + 2) % 4], isems[(i + 2) % 4])

                pltpu.make_async_copy(xc_sh.at[ib.at[0]], rb,
                                      gsems[i % 2]).wait()

                @pl.when(cc + 1 < K)
                def _():
                    wait_idx(idxs[(i + 1) % 4], isems[(i + 1) % 4])
                    pltpu.async_copy(xc_sh.at[idxs[(i + 1) % 4].at[0]],
                                     rows[(i + 1) % 2], gsems[(i + 1) % 2])

                pltpu.sync_copy(rb, acc_sh.at[ib.at[1]], add=True)

                @pl.when(c == 0)
                def _():
                    @pl.loop(0, CHUNK, step=16)
                    def _(t):
                        plsc.addupdate_scatter(cnt_l, [ib[1, pl.ds(t, 16)]],
                                               ones16)

        toff = ebase + K * CHUNK
        pltpu.sync_copy(e_hbm.at[0, pl.ds(toff, TAIL)],
                        i0.at[0, pl.ds(0, TAIL)])
        pltpu.sync_copy(e_hbm.at[1, pl.ds(toff, TAIL)],
                        i0.at[1, pl.ds(0, TAIL)])

        @pl.loop(TAIL, CHUNK, step=16)
        def _(t):
            i0[0, pl.ds(t, 16)] = jnp.zeros((16,), jnp.int32)
            i0[1, pl.ds(t, 16)] = jnp.full((16,), N_NODES, jnp.int32)

        pltpu.sync_copy(xc_sh.at[i0.at[0]], rows0)
        pltpu.sync_copy(rows0, acc_sh.at[i0.at[1]], add=True)

        @pl.when(c == 0)
        def _():
            @pl.loop(0, TAIL, step=16)
            def _(t):
                plsc.addupdate_scatter(cnt_l, [i0[1, pl.ds(t, 16)]], ones16)

        plsc.subcore_barrier()

        pltpu.sync_copy(acc_sh.at[pl.ds(base, ROWS_PER_SUB)],
                        acc_hbm.at[c, pl.ds(base, ROWS_PER_SUB)])

        @pl.when(c == 0)
        def _():
            pltpu.sync_copy(cnt_l, cnt_hbm.at[s])

    return k(x, e2)


def _combine_body(x_ref, a_ref, c_ref, o_ref):
    xb = x_ref[...]
    inv = 1.0 / (jnp.sum(c_ref[...], axis=1, keepdims=True) + 1.0)
    o_ref[:, :D_FEAT] = xb
    o_ref[:, D_FEAT:D_FEAT + DH] = (a_ref[0] + xb[:, :DH]) * inv
    o_ref[:, D_FEAT + DH:] = (a_ref[1] + xb[:, DH:]) * inv


def _tc_combine(x, acc, cnt):
    blk = 1000
    return pl.pallas_call(
        _combine_body,
        out_shape=jax.ShapeDtypeStruct((N_NODES, 2 * D_FEAT), jnp.float32),
        grid=(N_NODES // blk,),
        in_specs=[
            pl.BlockSpec((blk, D_FEAT), lambda i: (i, 0)),
            pl.BlockSpec((NC, blk, DH), lambda i: (0, i, 0)),
            pl.BlockSpec((blk, NS), lambda i: (i, 0)),
        ],
        out_specs=pl.BlockSpec((blk, 2 * D_FEAT), lambda i: (i, 0)),
    )(x, acc, cnt)


def kernel(x, edge_index):
    e2 = edge_index.astype(jnp.int32)
    acc, cnt = _sc_segment_sum(x, e2)
    return _tc_combine(x, acc, cnt.T)
